# baseline (speedup 1.0000x reference)
import functools

import jax
import jax.numpy as jnp
from jax import lax
from jax.experimental import pallas as pl
from jax.experimental.pallas import tpu as pltpu
from jax.experimental.pallas import tpu_sc as plsc

_VOCAB = 2048
_DIM = 2048
_SEQ = 50
_NCLASS = 20
_CW = 20
_STRIDE = 21
_ROWS = _VOCAB + 8
_NC = 2
_NS = 16
_NW = _NC * _NS
_L = 16


def _project_body(table_ref, w1_ref, w2_ref, b1_ref, b2_ref, out_ref):
    t = table_ref[...]
    h = lax.dot_general(t, w1_ref[...], (((1,), (1,)), ((), ())),
                        preferred_element_type=jnp.float32)
    proj = lax.dot_general(h, w2_ref[...], (((1,), (1,)), ((), ())),
                           preferred_element_type=jnp.float32)
    out_ref[0:_VOCAB, 0:_CW] = proj * (1.0 / _SEQ)
    brow = lax.dot_general(b1_ref[...], w2_ref[...], (((1,), (1,)), ((), ())),
                           preferred_element_type=jnp.float32) + b2_ref[...]
    out_ref[_VOCAB:_ROWS, 0:_CW] = jnp.broadcast_to(brow, (_ROWS - _VOCAB, _CW))


def _project_table(table, w1, w2, b1, b2):
    return pl.pallas_call(
        _project_body,
        out_shape=jax.ShapeDtypeStruct((_ROWS, _STRIDE), jnp.float32),
    )(table, w1, w2, b1.reshape(1, -1), b2.reshape(1, -1))


def _sc_pool(tblp, x, s, bpw):
    mesh = plsc.VectorSubcoreMesh(core_axis_name="c", subcore_axis_name="s")
    groups = bpw // _L

    @functools.partial(
        pl.kernel,
        mesh=mesh,
        out_type=jax.ShapeDtypeStruct((_NW * bpw, _CW), jnp.float32),
        compiler_params=pltpu.CompilerParams(needs_layout_passes=False,
                                             use_tc_tiling_on_sc=False),
        scratch_types=[
            pltpu.VMEM((_ROWS, _STRIDE), jnp.float32),
            pltpu.VMEM((bpw, s), jnp.int32),
            pltpu.VMEM((bpw, _CW), jnp.float32),
        ],
    )
    def pool(tbl_hbm, x_hbm, out_hbm, tbl_v, idx_v, outb_v):
        wid = lax.axis_index("s") * _NC + lax.axis_index("c")
        pltpu.sync_copy(tbl_hbm, tbl_v)
        pltpu.sync_copy(x_hbm.at[pl.ds(wid * bpw, bpw)], idx_v)
        lane = jnp.arange(_L, dtype=jnp.int32)
        bias_rows = jnp.full((_L,), _VOCAB, jnp.int32)
        init = tuple(
            plsc.load_gather(tbl_v, [bias_rows, jnp.full((_L,), c, jnp.int32)])
            for c in range(_CW))
        for g in range(groups):
            lrows = lane + (g * _L)
            def body(i, carry, _lrows=lrows):
                rows = plsc.load_gather(idx_v, [_lrows, jnp.full((_L,), 0, jnp.int32) + i])
                return tuple(
                    carry[c] + plsc.load_gather(
                        tbl_v, [rows, jnp.full((_L,), c, jnp.int32)])
                    for c in range(_CW))
            acc = lax.fori_loop(0, s, body, init)
            for c in range(_CW):
                plsc.store_scatter(outb_v, [lrows, jnp.full((_L,), c, jnp.int32)],
                                   acc[c])
        pltpu.sync_copy(outb_v, out_hbm.at[pl.ds(wid * bpw, bpw)])

    return pool(tblp, x)


def kernel(x, table, W1, b1, W2, b2):
    b, s = x.shape
    bpw = b // _NW
    tblp = _project_table(table, W1, W2, b1, b2)
    return _sc_pool(tblp, x, s, bpw)


# speedup vs baseline: 1.0154x; 1.0154x over previous
import functools

import jax
import jax.numpy as jnp
from jax import lax
from jax.experimental import pallas as pl
from jax.experimental.pallas import tpu as pltpu
from jax.experimental.pallas import tpu_sc as plsc

_VOCAB = 2048
_DIM = 2048
_SEQ = 50
_NCLASS = 20
_CW = 20
_CH = _VOCAB // 128
_CLS = 24
_NC = 2
_NS = 16
_NW = _NC * _NS
_L = 16


def _project_body(table_ref, w1_ref, w2_ref, b1_ref, b2_ref, out_ref, bias_ref):
    w21 = lax.dot_general(w2_ref[...], w1_ref[...], (((1,), (0,)), ((), ())),
                          preferred_element_type=jnp.float32)
    projt = lax.dot_general(w21, table_ref[...], (((1,), (1,)), ((), ())),
                            preferred_element_type=jnp.float32)
    projt = projt * (1.0 / _SEQ)
    for k in range(_CH):
        out_ref[k, 0:_CW, :] = projt[:, k * 128:(k + 1) * 128]
    brow = lax.dot_general(b1_ref[...], w2_ref[...], (((1,), (1,)), ((), ())),
                           preferred_element_type=jnp.float32) + b2_ref[...]
    bias_ref[0:1, 0:_CW] = brow


def _project_table(table, w1, w2, b1, b2):
    return pl.pallas_call(
        _project_body,
        out_shape=(jax.ShapeDtypeStruct((_CH, _CLS, 128), jnp.float32),
                   jax.ShapeDtypeStruct((8, 128), jnp.float32)),
    )(table, w1, w2, b1.reshape(1, -1), b2.reshape(1, -1))


def _sc_pool(tblp, bias, x, s, bpw):
    mesh = plsc.VectorSubcoreMesh(core_axis_name="c", subcore_axis_name="s")
    groups = bpw // _L

    @functools.partial(
        pl.kernel,
        mesh=mesh,
        out_type=jax.ShapeDtypeStruct((_NW * bpw, _CW), jnp.float32),
        compiler_params=pltpu.CompilerParams(needs_layout_passes=False),
        scratch_types=[
            pltpu.VMEM((_CH, _CLS, 128), jnp.float32),
            pltpu.VMEM((8, 128), jnp.float32),
            pltpu.VMEM((bpw, s), jnp.int32),
            pltpu.VMEM((bpw, _CW), jnp.float32),
        ],
    )
    def pool(tbl_hbm, bias_hbm, x_hbm, out_hbm, tbl_v, bias_v, idx_v, outb_v):
        wid = lax.axis_index("s") * _NC + lax.axis_index("c")
        pltpu.sync_copy(tbl_hbm, tbl_v)
        pltpu.sync_copy(bias_hbm, bias_v)
        pltpu.sync_copy(x_hbm.at[pl.ds(wid * bpw, bpw)], idx_v)
        lane = jnp.arange(_L, dtype=jnp.int32)
        lane3 = lane * 3
        zero = jnp.zeros((_L,), jnp.int32)
        init = tuple(
            plsc.load_gather(bias_v, [zero, jnp.full((_L,), c, jnp.int32)])
            for c in range(_CW))
        for g in range(groups):
            lrows = lane + (g * _L)
            def body(i, carry, _lrows=lrows):
                scol = lax.rem(lane3 + i, s)
                rows = plsc.load_gather(idx_v, [_lrows, scol])
                ch = lax.shift_right_logical(rows, 7)
                ln = lax.bitwise_and(rows, 127)
                return tuple(
                    carry[c] + plsc.load_gather(
                        tbl_v, [ch, jnp.full((_L,), c, jnp.int32), ln])
                    for c in range(_CW))
            acc = lax.fori_loop(0, s, body, init)
            for c in range(_CW):
                plsc.store_scatter(outb_v, [lrows, jnp.full((_L,), c, jnp.int32)],
                                   acc[c])
        pltpu.sync_copy(outb_v, out_hbm.at[pl.ds(wid * bpw, bpw)])

    return pool(tblp, bias, x)


def kernel(x, table, W1, b1, W2, b2):
    b, s = x.shape
    bpw = b // _NW
    tblp, bias = _project_table(table, W1, W2, b1, b2)
    return _sc_pool(tblp, bias, x, s, bpw)


# speedup vs baseline: 1.0256x; 1.0100x over previous
import functools

import jax
import jax.numpy as jnp
from jax import lax
from jax.experimental import pallas as pl
from jax.experimental.pallas import tpu as pltpu
from jax.experimental.pallas import tpu_sc as plsc

_VOCAB = 2048
_DIM = 2048
_SEQ = 50
_NCLASS = 20
_CW = 20
_CH = _VOCAB // 128
_CLS = 24
_NC = 2
_NS = 16
_NW = _NC * _NS
_L = 16


def _project_body(table_ref, w1_ref, w2_ref, b1_ref, b2_ref, out_ref, bias_ref):
    w21 = lax.dot_general(w2_ref[...], w1_ref[...], (((1,), (0,)), ((), ())),
                          preferred_element_type=jnp.float32)
    projt = lax.dot_general(w21, table_ref[...], (((1,), (1,)), ((), ())),
                            preferred_element_type=jnp.float32)
    projt = projt * (1.0 / _SEQ)
    for k in range(_CH):
        out_ref[k, 0:_CW, :] = projt[:, k * 128:(k + 1) * 128]
    brow = lax.dot_general(b1_ref[...], w2_ref[...], (((1,), (1,)), ((), ())),
                           preferred_element_type=jnp.float32) + b2_ref[...]
    bias_ref[0:1, 0:_CW] = brow


def _project_table(table, w1, w2, b1, b2):
    return pl.pallas_call(
        _project_body,
        out_shape=(jax.ShapeDtypeStruct((_CH, _CLS, 128), jnp.float32),
                   jax.ShapeDtypeStruct((8, 128), jnp.float32)),
    )(table, w1, w2, b1.reshape(1, -1), b2.reshape(1, -1))


def _sc_pool(tblp, bias, x, s, bpw):
    mesh = plsc.VectorSubcoreMesh(core_axis_name="c", subcore_axis_name="s")
    groups = bpw // _L

    @functools.partial(
        pl.kernel,
        mesh=mesh,
        out_type=jax.ShapeDtypeStruct((_NW * bpw, _CW), jnp.float32),
        compiler_params=pltpu.CompilerParams(needs_layout_passes=False),
        scratch_types=[
            pltpu.VMEM((_CH, _CLS, 128), jnp.float32),
            pltpu.VMEM((8, 128), jnp.float32),
            pltpu.VMEM((bpw, s), jnp.int32),
            pltpu.VMEM((bpw, _CW), jnp.float32),
        ],
    )
    def pool(tbl_hbm, bias_hbm, x_hbm, out_hbm, tbl_v, bias_v, idx_v, outb_v):
        wid = lax.axis_index("s") * _NC + lax.axis_index("c")
        pltpu.sync_copy(tbl_hbm, tbl_v)
        pltpu.sync_copy(bias_hbm, bias_v)
        pltpu.sync_copy(x_hbm.at[pl.ds(wid * bpw, bpw)], idx_v)
        lane = jnp.arange(_L, dtype=jnp.int32)
        lane3 = lane * 3
        zero = jnp.zeros((_L,), jnp.int32)
        init = tuple(
            plsc.load_gather(bias_v, [zero, jnp.full((_L,), c, jnp.int32)])
            for c in range(_CW))
        for g in range(groups):
            lrows = lane + (g * _L)
            def body(i, carry, _lrows=lrows):
                sraw = lane3 + i
                scol = jnp.where(sraw >= s, sraw - s, sraw)
                rows = plsc.load_gather(idx_v, [_lrows, scol])
                ch = lax.shift_right_logical(rows, 7)
                ln = lax.bitwise_and(rows, 127)
                return tuple(
                    carry[c] + plsc.load_gather(
                        tbl_v, [ch, jnp.full((_L,), c, jnp.int32), ln])
                    for c in range(_CW))
            acc = lax.fori_loop(0, s, body, init)
            for c in range(_CW):
                plsc.store_scatter(outb_v, [lrows, jnp.full((_L,), c, jnp.int32)],
                                   acc[c])
        pltpu.sync_copy(outb_v, out_hbm.at[pl.ds(wid * bpw, bpw)])

    return pool(tblp, bias, x)


def kernel(x, table, W1, b1, W2, b2):
    b, s = x.shape
    bpw = b // _NW
    tblp, bias = _project_table(table, W1, W2, b1, b2)
    return _sc_pool(tblp, bias, x, s, bpw)


# speedup vs baseline: 1.0343x; 1.0085x over previous
import functools

import jax
import jax.numpy as jnp
from jax import lax
from jax.experimental import pallas as pl
from jax.experimental.pallas import tpu as pltpu
from jax.experimental.pallas import tpu_sc as plsc

_VOCAB = 2048
_DIM = 2048
_SEQ = 50
_NCLASS = 20
_CW = 20
_CH = _VOCAB // 128
_CLS = 24
_NC = 2
_NS = 16
_NW = _NC * _NS
_L = 16


def _project_body(table_ref, w1_ref, w2_ref, b1_ref, b2_ref, x_ref,
                  out_ref, bias_ref, xlin_ref):
    w21 = lax.dot_general(w2_ref[...], w1_ref[...], (((1,), (0,)), ((), ())),
                          preferred_element_type=jnp.float32)
    projt = lax.dot_general(w21, table_ref[...], (((1,), (1,)), ((), ())),
                            preferred_element_type=jnp.float32)
    projt = projt * (1.0 / _SEQ)
    for k in range(_CH):
        out_ref[k, 0:_CW, :] = projt[:, k * 128:(k + 1) * 128]
    brow = lax.dot_general(b1_ref[...], w2_ref[...], (((1,), (1,)), ((), ())),
                           preferred_element_type=jnp.float32) + b2_ref[...]
    bias_ref[0:1, 0:_CW] = brow
    xlin_ref[:, 0:_SEQ] = x_ref[...]


def _project_table(table, w1, w2, b1, b2, x):
    return pl.pallas_call(
        _project_body,
        out_shape=(jax.ShapeDtypeStruct((_CH, _CLS, 128), jnp.float32),
                   jax.ShapeDtypeStruct((8, 128), jnp.float32),
                   jax.ShapeDtypeStruct((x.shape[0], 128), jnp.int32)),
    )(table, w1, w2, b1.reshape(1, -1), b2.reshape(1, -1), x)


def _sc_pool(tblp, bias, x, s, bpw):
    mesh = plsc.VectorSubcoreMesh(core_axis_name="c", subcore_axis_name="s")
    groups = bpw // _L

    @functools.partial(
        pl.kernel,
        mesh=mesh,
        out_type=jax.ShapeDtypeStruct((_NW * bpw, _CW), jnp.float32),
        compiler_params=pltpu.CompilerParams(needs_layout_passes=False),
        scratch_types=[
            pltpu.VMEM((_CH, _CLS, 128), jnp.float32),
            pltpu.VMEM((8, 128), jnp.float32),
            pltpu.VMEM((bpw, 128), jnp.int32),
            pltpu.VMEM((bpw, _CW), jnp.float32),
            pltpu.SemaphoreType.DMA,
            pltpu.SemaphoreType.DMA,
            pltpu.SemaphoreType.DMA,
        ],
    )
    def pool(tbl_hbm, bias_hbm, x_hbm, out_hbm, tbl_v, bias_v, idx_v, outb_v,
             sem0, sem1, sem2):
        wid = lax.axis_index("s") * _NC + lax.axis_index("c")
        cp0 = pltpu.async_copy(tbl_hbm, tbl_v, sem0)
        cp1 = pltpu.async_copy(bias_hbm, bias_v, sem1)
        cp2 = pltpu.async_copy(x_hbm.at[pl.ds(wid * bpw, bpw)], idx_v, sem2)
        cp0.wait()
        cp1.wait()
        cp2.wait()
        lane = jnp.arange(_L, dtype=jnp.int32)
        lane3 = lane * 3
        zero = jnp.zeros((_L,), jnp.int32)
        init = tuple(
            plsc.load_gather(bias_v, [zero, jnp.full((_L,), c, jnp.int32)])
            for c in range(_CW))
        for g in range(groups):
            lrows = lane + (g * _L)
            def body(i, carry, _lrows=lrows):
                sraw = lane3 + i
                scol = jnp.where(sraw >= s, sraw - s, sraw)
                rows = plsc.load_gather(idx_v, [_lrows, scol])
                ch = lax.shift_right_logical(rows, 7)
                ln = lax.bitwise_and(rows, 127)
                return tuple(
                    carry[c] + plsc.load_gather(
                        tbl_v, [ch, jnp.full((_L,), c, jnp.int32), ln])
                    for c in range(_CW))
            acc = lax.fori_loop(0, s, body, init)
            for c in range(_CW):
                plsc.store_scatter(outb_v, [lrows, jnp.full((_L,), c, jnp.int32)],
                                   acc[c])
        pltpu.sync_copy(outb_v, out_hbm.at[pl.ds(wid * bpw, bpw)])

    return pool(tblp, bias, x)


def kernel(x, table, W1, b1, W2, b2):
    b, s = x.shape
    bpw = b // _NW
    tblp, bias, xlin = _project_table(table, W1, W2, b1, b2, x)
    return _sc_pool(tblp, bias, xlin, s, bpw)


# speedup vs baseline: 1.1250x; 1.0877x over previous
import functools

import jax
import jax.numpy as jnp
from jax import lax
from jax.experimental import pallas as pl
from jax.experimental.pallas import tpu as pltpu
from jax.experimental.pallas import tpu_sc as plsc

_VOCAB = 2048
_DIM = 2048
_SEQ = 50
_NCLASS = 20
_CW = 20
_CH = _VOCAB // 128
_CLS = 24
_NC = 2
_NS = 16
_NW = _NC * _NS
_L = 16


_SP = (_SEQ + 7) // 8 * 8


def _project_body(table_ref, w1_ref, w2_ref, b1_ref, b2_ref, xt_ref,
                  out_ref, bias_ref, xb_ref):
    w21 = lax.dot_general(w2_ref[...], w1_ref[...], (((1,), (0,)), ((), ())),
                          preferred_element_type=jnp.float32)
    projt = lax.dot_general(w21, table_ref[...], (((1,), (1,)), ((), ())),
                            preferred_element_type=jnp.float32)
    projt = projt * (1.0 / _SEQ)
    for k in range(_CH):
        out_ref[k, 0:_CW, :] = projt[:, k * 128:(k + 1) * 128]
    brow = lax.dot_general(b1_ref[...], w2_ref[...], (((1,), (1,)), ((), ())),
                           preferred_element_type=jnp.float32) + b2_ref[...]
    bias_ref[0:1, 0:_CW] = brow
    for w in range(_NW):
        xb_ref[w, 0:_SEQ, :] = xt_ref[:, w * 128:(w + 1) * 128]


def _project_table(table, w1, w2, b1, b2, xt):
    return pl.pallas_call(
        _project_body,
        out_shape=(jax.ShapeDtypeStruct((_CH, _CLS, 128), jnp.float32),
                   jax.ShapeDtypeStruct((8, 128), jnp.float32),
                   jax.ShapeDtypeStruct((_NW, _SP, 128), jnp.int32)),
    )(table, w1, w2, b1.reshape(1, -1), b2.reshape(1, -1), xt)


def _sc_pool(tblp, bias, x, s, bpw):
    mesh = plsc.VectorSubcoreMesh(core_axis_name="c", subcore_axis_name="s")
    groups = bpw // _L

    @functools.partial(
        pl.kernel,
        mesh=mesh,
        out_type=jax.ShapeDtypeStruct((_NW * bpw, _CW), jnp.float32),
        compiler_params=pltpu.CompilerParams(needs_layout_passes=False),
        scratch_types=[
            pltpu.VMEM((_CH, _CLS, 128), jnp.float32),
            pltpu.VMEM((8, 128), jnp.float32),
            pltpu.VMEM((_SP, 128), jnp.int32),
            pltpu.VMEM((bpw, _CW), jnp.float32),
            pltpu.SemaphoreType.DMA,
            pltpu.SemaphoreType.DMA,
            pltpu.SemaphoreType.DMA,
        ],
    )
    def pool(tbl_hbm, bias_hbm, x_hbm, out_hbm, tbl_v, bias_v, idx_v, outb_v,
             sem0, sem1, sem2):
        wid = lax.axis_index("s") * _NC + lax.axis_index("c")
        cp0 = pltpu.async_copy(tbl_hbm, tbl_v, sem0)
        cp1 = pltpu.async_copy(bias_hbm, bias_v, sem1)
        cp2 = pltpu.async_copy(x_hbm.at[wid], idx_v, sem2)
        cp0.wait()
        cp1.wait()
        cp2.wait()
        lane = jnp.arange(_L, dtype=jnp.int32)
        zero = jnp.zeros((_L,), jnp.int32)
        init = tuple(
            plsc.load_gather(bias_v, [zero, jnp.full((_L,), c, jnp.int32)])
            for c in range(_CW))
        for g in range(groups):
            lrows = lane + (g * _L)
            def body(i, carry, _g=g):
                rows = idx_v[i, pl.ds(_g * _L, _L)]
                ch = lax.shift_right_logical(rows, 7)
                ln = lax.bitwise_and(rows, 127)
                return tuple(
                    carry[c] + plsc.load_gather(
                        tbl_v, [ch, jnp.full((_L,), c, jnp.int32), ln])
                    for c in range(_CW))
            acc = lax.fori_loop(0, s, body, init)
            for c in range(_CW):
                plsc.store_scatter(outb_v, [lrows, jnp.full((_L,), c, jnp.int32)],
                                   acc[c])
        pltpu.sync_copy(outb_v, out_hbm.at[pl.ds(wid * bpw, bpw)])

    return pool(tblp, bias, x)


def kernel(x, table, W1, b1, W2, b2):
    b, s = x.shape
    bpw = b // _NW
    tblp, bias, xb = _project_table(table, W1, W2, b1, b2, x.T)
    return _sc_pool(tblp, bias, xb, s, bpw)


# speedup vs baseline: 1.1745x; 1.0440x over previous
import functools

import jax
import jax.numpy as jnp
from jax import lax
from jax.experimental import pallas as pl
from jax.experimental.pallas import tpu as pltpu
from jax.experimental.pallas import tpu_sc as plsc

_VOCAB = 2048
_DIM = 2048
_SEQ = 50
_NCLASS = 20
_CW = 20
_CH = _VOCAB // 128
_CLS = 24
_NC = 2
_NS = 16
_NW = _NC * _NS
_L = 16


_SP = (_SEQ + 7) // 8 * 8


def _project_body(table_ref, w1_ref, w2_ref, b1_ref, b2_ref, xt_ref,
                  out_ref, bias_ref, xb_ref):
    w21 = lax.dot_general(w2_ref[...], w1_ref[...], (((1,), (0,)), ((), ())),
                          preferred_element_type=jnp.float32)
    projt = lax.dot_general(w21, table_ref[...], (((1,), (1,)), ((), ())),
                            preferred_element_type=jnp.float32)
    projt = projt * (1.0 / _SEQ)
    for k in range(_CH):
        out_ref[k, 0:_CW, :] = projt[:, k * 128:(k + 1) * 128]
    brow = lax.dot_general(b1_ref[...], w2_ref[...], (((1,), (1,)), ((), ())),
                           preferred_element_type=jnp.float32) + b2_ref[...]
    bias_ref[0:1, 0:_CW] = brow
    for w in range(_NW):
        xb_ref[w, 0:_SEQ, :] = xt_ref[:, w * 128:(w + 1) * 128]


def _project_table(table, w1, w2, b1, b2, xt):
    return pl.pallas_call(
        _project_body,
        out_shape=(jax.ShapeDtypeStruct((_CH, _CLS, 128), jnp.float32),
                   jax.ShapeDtypeStruct((8, 128), jnp.float32),
                   jax.ShapeDtypeStruct((_NW, _SP, 128), jnp.int32)),
    )(table, w1, w2, b1.reshape(1, -1), b2.reshape(1, -1), xt)


def _sc_pool(tblp, bias, x, s, bpw):
    mesh = plsc.VectorSubcoreMesh(core_axis_name="c", subcore_axis_name="s")
    groups = bpw // _L

    unroll = 5 if s % 5 == 0 else 1

    @functools.partial(
        pl.kernel,
        mesh=mesh,
        out_type=jax.ShapeDtypeStruct((_CW, _NW * bpw), jnp.float32),
        compiler_params=pltpu.CompilerParams(needs_layout_passes=False),
        scratch_types=[
            pltpu.VMEM((_CH, _CLS, 128), jnp.float32),
            pltpu.VMEM((8, 128), jnp.float32),
            pltpu.VMEM((_SP, 128), jnp.int32),
            pltpu.VMEM((_CW, bpw), jnp.float32),
            pltpu.SemaphoreType.DMA,
            pltpu.SemaphoreType.DMA,
            pltpu.SemaphoreType.DMA,
        ],
    )
    def pool(tbl_hbm, bias_hbm, x_hbm, out_hbm, tbl_v, bias_v, idx_v, outb_v,
             sem0, sem1, sem2):
        wid = lax.axis_index("s") * _NC + lax.axis_index("c")
        cp0 = pltpu.async_copy(tbl_hbm, tbl_v, sem0)
        cp1 = pltpu.async_copy(bias_hbm, bias_v, sem1)
        cp2 = pltpu.async_copy(x_hbm.at[wid], idx_v, sem2)
        cp0.wait()
        cp1.wait()
        cp2.wait()
        zero = jnp.zeros((_L,), jnp.int32)
        init = tuple(
            plsc.load_gather(bias_v, [zero, jnp.full((_L,), c, jnp.int32)])
            for c in range(_CW))
        for g in range(groups):
            def body(i, carry, _g=g):
                for u in range(unroll):
                    rows = idx_v[i * unroll + u, pl.ds(_g * _L, _L)]
                    ch = lax.shift_right_logical(rows, 7)
                    ln = lax.bitwise_and(rows, 127)
                    carry = tuple(
                        carry[c] + plsc.load_gather(
                            tbl_v, [ch, jnp.full((_L,), c, jnp.int32), ln])
                        for c in range(_CW))
                return carry
            acc = lax.fori_loop(0, s // unroll, body, init)
            for c in range(_CW):
                outb_v[c, pl.ds(g * _L, _L)] = acc[c]
        pltpu.sync_copy(outb_v, out_hbm.at[:, pl.ds(wid * bpw, bpw)])

    return pool(tblp, bias, x)


def kernel(x, table, W1, b1, W2, b2):
    b, s = x.shape
    bpw = b // _NW
    tblp, bias, xb = _project_table(table, W1, W2, b1, b2, x.T)
    return _sc_pool(tblp, bias, xb, s, bpw).T


# speedup vs baseline: 1.2746x; 1.0852x over previous
import functools

import jax
import jax.numpy as jnp
from jax import lax
from jax.experimental import pallas as pl
from jax.experimental.pallas import tpu as pltpu
from jax.experimental.pallas import tpu_sc as plsc

_VOCAB = 2048
_DIM = 2048
_SEQ = 50
_NCLASS = 20
_CW = 20
_CH = _VOCAB // 128
_CLS = 24
_NC = 2
_NS = 16
_NW = _NC * _NS
_L = 16


_SP = (_SEQ + 7) // 8 * 8


_CP = _CW // 2
_CPS = 16


def _project_body(table_ref, w1_ref, w2_ref, b1_ref, b2_ref, xt_ref,
                  out_ref, bias_ref, xb_ref):
    w21 = lax.dot_general(w2_ref[...], w1_ref[...], (((1,), (0,)), ((), ())),
                          preferred_element_type=jnp.float32)
    projt = lax.dot_general(w21, table_ref[...], (((1,), (1,)), ((), ())),
                            preferred_element_type=jnp.float32)
    projt = projt * (1.0 / _SEQ)
    bits = lax.bitcast_convert_type(projt, jnp.int32)
    r = lax.shift_right_logical(bits + 0x8000, 16)
    packed = jnp.bitwise_or(r[0:_CP, :],
                            lax.shift_left(r[_CP:_CW, :], 16))
    for k in range(_CH):
        out_ref[k, 0:_CP, :] = packed[:, k * 128:(k + 1) * 128]
    brow = lax.dot_general(b1_ref[...], w2_ref[...], (((1,), (1,)), ((), ())),
                           preferred_element_type=jnp.float32) + b2_ref[...]
    bias_ref[0:1, 0:_CW] = brow
    for w in range(_NW):
        xb_ref[w, 0:_SEQ, :] = xt_ref[:, w * 128:(w + 1) * 128]


def _project_table(table, w1, w2, b1, b2, xt):
    return pl.pallas_call(
        _project_body,
        out_shape=(jax.ShapeDtypeStruct((_CH, _CPS, 128), jnp.int32),
                   jax.ShapeDtypeStruct((8, 128), jnp.float32),
                   jax.ShapeDtypeStruct((_NW, _SP, 128), jnp.int32)),
    )(table, w1, w2, b1.reshape(1, -1), b2.reshape(1, -1), xt)


def _sc_pool(tblp, bias, x, s, bpw):
    mesh = plsc.VectorSubcoreMesh(core_axis_name="c", subcore_axis_name="s")
    groups = bpw // _L

    unroll = 5 if s % 5 == 0 else 1

    @functools.partial(
        pl.kernel,
        mesh=mesh,
        out_type=jax.ShapeDtypeStruct((_CW, _NW * bpw), jnp.float32),
        compiler_params=pltpu.CompilerParams(needs_layout_passes=False),
        scratch_types=[
            pltpu.VMEM((_CH, _CPS, 128), jnp.int32),
            pltpu.VMEM((8, 128), jnp.float32),
            pltpu.VMEM((_SP, 128), jnp.int32),
            pltpu.VMEM((_CW, bpw), jnp.float32),
            pltpu.SemaphoreType.DMA,
            pltpu.SemaphoreType.DMA,
            pltpu.SemaphoreType.DMA,
        ],
    )
    def pool(tbl_hbm, bias_hbm, x_hbm, out_hbm, tbl_v, bias_v, idx_v, outb_v,
             sem0, sem1, sem2):
        wid = lax.axis_index("s") * _NC + lax.axis_index("c")
        cp0 = pltpu.async_copy(tbl_hbm, tbl_v, sem0)
        cp1 = pltpu.async_copy(bias_hbm, bias_v, sem1)
        cp2 = pltpu.async_copy(x_hbm.at[wid], idx_v, sem2)
        cp0.wait()
        cp1.wait()
        cp2.wait()
        zero = jnp.zeros((_L,), jnp.int32)
        init = tuple(
            plsc.load_gather(bias_v, [zero, jnp.full((_L,), c, jnp.int32)])
            for c in range(_CW))
        for g in range(groups):
            def body(i, carry, _g=g):
                for u in range(unroll):
                    rows = idx_v[i * unroll + u, pl.ds(_g * _L, _L)]
                    ch = lax.shift_right_logical(rows, 7)
                    ln = lax.bitwise_and(rows, 127)
                    words = [plsc.load_gather(
                        tbl_v, [ch, jnp.full((_L,), j, jnp.int32), ln])
                        for j in range(_CP)]
                    lo = [plsc.bitcast(lax.shift_left(w, 16), jnp.float32)
                          for w in words]
                    hi = [plsc.bitcast(
                        lax.bitwise_and(w, jnp.int32(-65536)), jnp.float32)
                        for w in words]
                    carry = tuple(carry[c] + lo[c] for c in range(_CP)) + \
                            tuple(carry[_CP + c] + hi[c] for c in range(_CP))
                return carry
            acc = lax.fori_loop(0, s // unroll, body, init)
            for c in range(_CW):
                outb_v[c, pl.ds(g * _L, _L)] = acc[c]
        pltpu.sync_copy(outb_v, out_hbm.at[:, pl.ds(wid * bpw, bpw)])

    return pool(tblp, bias, x)


def kernel(x, table, W1, b1, W2, b2):
    b, s = x.shape
    bpw = b // _NW
    tblp, bias, xb = _project_table(table, W1, W2, b1, b2, x.T)
    return _sc_pool(tblp, bias, xb, s, bpw).T
